# baseline (speedup 1.0000x reference)
import functools

import jax
import jax.numpy as jnp
from jax import lax
from jax.experimental import pallas as pl
from jax.experimental.pallas import tpu as pltpu
from jax.experimental.pallas import tpu_sc as plsc

NC = 2
NS = 16
NW = NC * NS
LANES = 16
K = 128


def _sc_mesh():
  return plsc.VectorSubcoreMesh(
      core_axis_name="c", subcore_axis_name="s",
      num_cores=NC, num_subcores=NS)


def _deg_hist(dst_pad, zeros_d, ones_d, a_rows, rpt, cpw, epw, d):
  @functools.partial(
      pl.kernel,
      out_type=jax.ShapeDtypeStruct((NC, a_rows, d), jnp.float32),
      mesh=_sc_mesh(),
      scratch_types=[
          pltpu.VMEM((K,), jnp.int32),
          pltpu.VMEM((K, d), jnp.float32),
          pltpu.VMEM_SHARED((a_rows, d), jnp.float32),
      ])
  def body(dst_hbm, z_hbm, ones_hbm, out_hbm, didx, ones_v, acc):
    c = lax.axis_index("c")
    s = lax.axis_index("s")
    wid = c * NS + s
    pltpu.sync_copy(z_hbm.at[pl.ds(s * rpt, rpt)], acc.at[pl.ds(s * rpt, rpt)])
    pltpu.sync_copy(ones_hbm, ones_v)
    plsc.subcore_barrier()
    base = wid * epw

    def step(j, carry):
      off = base + j * K
      pltpu.sync_copy(dst_hbm.at[pl.ds(off, K)], didx)
      pltpu.sync_copy(ones_v, acc.at[didx], add=True)
      return carry

    lax.fori_loop(0, cpw, step, 0)
    plsc.subcore_barrier()
    pltpu.sync_copy(acc.at[pl.ds(s * rpt, rpt)],
                    out_hbm.at[c, pl.ds(s * rpt, rpt)])

  return body(dst_pad, zeros_d, ones_d)


SCK = 2


def _sc_scatter(g, src2, dst2, zeros_d, a_rows, rpt, cpw, epw):
  d = g.shape[1]
  assert cpw % SCK == 0

  @functools.partial(
      pl.kernel,
      out_type=jax.ShapeDtypeStruct((NC, a_rows, d), jnp.float32),
      mesh=_sc_mesh(),
      scratch_types=[
          pltpu.VMEM((SCK, K), jnp.int32),
          pltpu.VMEM((SCK, K), jnp.int32),
          pltpu.VMEM((SCK * K, d), jnp.float32),
          pltpu.VMEM_SHARED((a_rows, d), jnp.float32),
          pltpu.SemaphoreType.DMA,
      ])
  def body(g_hbm, src_hbm, dst_hbm, z_hbm, out_hbm, sidx, didx, rows, acc, sem):
    c = lax.axis_index("c")
    s = lax.axis_index("s")
    wid = c * NS + s
    pltpu.sync_copy(z_hbm.at[pl.ds(s * rpt, rpt)], acc.at[pl.ds(s * rpt, rpt)])
    plsc.subcore_barrier()
    cbase = wid * cpw

    def step(t, carry):
      cb = cbase + t * SCK
      pltpu.sync_copy(src_hbm.at[pl.ds(cb, SCK)], sidx)
      pltpu.sync_copy(dst_hbm.at[pl.ds(cb, SCK)], didx)
      for i in range(SCK):
        pltpu.async_copy(g_hbm.at[sidx.at[i]], rows.at[pl.ds(i * K, K)], sem)
      for i in range(SCK):
        pltpu.make_async_copy(g_hbm.at[sidx.at[i]],
                              rows.at[pl.ds(i * K, K)], sem).wait()
      for i in range(SCK):
        pltpu.sync_copy(rows.at[pl.ds(i * K, K)], acc.at[didx.at[i]], add=True)
      return carry

    lax.fori_loop(0, cpw // SCK, step, 0)
    plsc.subcore_barrier()
    pltpu.sync_copy(acc.at[pl.ds(s * rpt, rpt)],
                    out_hbm.at[c, pl.ds(s * rpt, rpt)])

  return body(g, src2, dst2, zeros_d)


def _tc_first(degp, x, w1, r):
  n, d_in = x.shape
  d_hid = w1.shape[1]
  grid = (n // r,)

  def body(dp_ref, x_ref, w_ref, dinv_ref, g_ref):
    dp = dp_ref[...]
    deg = dp[0, :, 0:1] + dp[1, :, 0:1] + 1.0
    dinv = lax.rsqrt(deg)
    dinv_ref[...] = dinv
    g_ref[...] = dinv * jnp.dot(x_ref[...], w_ref[...],
                                preferred_element_type=jnp.float32)

  return pl.pallas_call(
      body,
      grid=grid,
      in_specs=[
          pl.BlockSpec((NC, r, d_hid), lambda i: (0, i, 0)),
          pl.BlockSpec((r, d_in), lambda i: (i, 0)),
          pl.BlockSpec((d_in, d_hid), lambda i: (0, 0)),
      ],
      out_specs=[
          pl.BlockSpec((r, 1), lambda i: (i, 0)),
          pl.BlockSpec((r, d_hid), lambda i: (i, 0)),
      ],
      out_shape=[
          jax.ShapeDtypeStruct((n, 1), jnp.float32),
          jax.ShapeDtypeStruct((n, d_hid), jnp.float32),
      ])(degp, x, w1)


def _tc_mid(part, g1, dinv, b1, w2, r):
  n, d = g1.shape
  a_rows = part.shape[1]
  grid = (n // r,)

  def body(p_ref, g_ref, dinv_ref, b_ref, w_ref, out_ref):
    p = p_ref[...]
    s = p[0] + p[1] + g_ref[...]
    h = dinv_ref[...] * s + b_ref[...]
    h = jnp.maximum(h, 0.0)
    out_ref[...] = dinv_ref[...] * jnp.dot(h, w_ref[...],
                                           preferred_element_type=jnp.float32)

  return pl.pallas_call(
      body,
      grid=grid,
      in_specs=[
          pl.BlockSpec((NC, r, d), lambda i: (0, i, 0)),
          pl.BlockSpec((r, d), lambda i: (i, 0)),
          pl.BlockSpec((r, 1), lambda i: (i, 0)),
          pl.BlockSpec((1, d), lambda i: (0, 0)),
          pl.BlockSpec((d, d), lambda i: (0, 0)),
      ],
      out_specs=pl.BlockSpec((r, d), lambda i: (i, 0)),
      out_shape=jax.ShapeDtypeStruct((n, d), jnp.float32))(
          part, g1, dinv, b1, w2)


def _tc_last(part, g2, dinv, b2, r):
  n, d = g2.shape
  grid = (n // r,)

  def body(p_ref, g_ref, dinv_ref, b_ref, out_ref):
    p = p_ref[...]
    s = p[0] + p[1] + g_ref[...]
    out_ref[...] = dinv_ref[...] * s + b_ref[...]

  return pl.pallas_call(
      body,
      grid=grid,
      in_specs=[
          pl.BlockSpec((NC, r, d), lambda i: (0, i, 0)),
          pl.BlockSpec((r, d), lambda i: (i, 0)),
          pl.BlockSpec((r, 1), lambda i: (i, 0)),
          pl.BlockSpec((1, d), lambda i: (0, 0)),
      ],
      out_specs=pl.BlockSpec((r, d), lambda i: (i, 0)),
      out_shape=jax.ShapeDtypeStruct((n, d), jnp.float32))(
          part, g2, dinv, b2)


def kernel(x, edge_index, W1, b1, W2, b2):
  n, d_in = x.shape
  d_hid = W1.shape[1]
  e = edge_index.shape[1]

  cpw = SCK * -(-e // (NW * K * SCK))
  epw = cpw * K
  e_pad = epw * NW
  pad = e_pad - e

  rpt = 632
  a_rows = rpt * NS

  src_pad = jnp.concatenate(
      [edge_index[0], jnp.zeros((pad,), jnp.int32)])
  dst_pad = jnp.concatenate(
      [edge_index[1], jnp.full((pad,), n, jnp.int32)])
  src2 = src_pad.reshape(-1, K)
  dst2 = dst_pad.reshape(-1, K)
  ones_d = jnp.ones((K, d_hid), jnp.float32)
  zeros_d = jnp.zeros((a_rows, d_hid), jnp.float32)

  r = 1000

  degp = _deg_hist(dst_pad, zeros_d, ones_d, a_rows, rpt, cpw, epw, d_hid)
  dinv, g1 = _tc_first(degp, x, W1, r)
  part1 = _sc_scatter(g1, src2, dst2, zeros_d, a_rows, rpt, cpw, epw)
  g2 = _tc_mid(part1, g1, dinv, b1.reshape(1, -1), W2, r)
  part2 = _sc_scatter(g2, src2, dst2, zeros_d, a_rows, rpt, cpw, epw)
  out = _tc_last(part2, g2, dinv, b2.reshape(1, -1), r)
  return out


# speedup vs baseline: 1.6002x; 1.6002x over previous
import functools

import jax
import jax.numpy as jnp
from jax import lax
from jax.experimental import pallas as pl
from jax.experimental.pallas import tpu as pltpu
from jax.experimental.pallas import tpu_sc as plsc

NC = 2
NS = 16
NW = NC * NS
LANES = 16
K = 128


def _sc_mesh():
  return plsc.VectorSubcoreMesh(
      core_axis_name="c", subcore_axis_name="s",
      num_cores=NC, num_subcores=NS)


def _deg_hist(dst_pad, zeros_d, ones_d, a_rows, rpt, cpw, epw, d):
  @functools.partial(
      pl.kernel,
      out_type=jax.ShapeDtypeStruct((NC, a_rows, d), jnp.float32),
      mesh=_sc_mesh(),
      scratch_types=[
          pltpu.VMEM((K,), jnp.int32),
          pltpu.VMEM((K, d), jnp.float32),
          pltpu.VMEM_SHARED((a_rows, d), jnp.float32),
      ])
  def body(dst_hbm, z_hbm, ones_hbm, out_hbm, didx, ones_v, acc):
    c = lax.axis_index("c")
    s = lax.axis_index("s")
    wid = c * NS + s
    pltpu.sync_copy(z_hbm.at[pl.ds(s * rpt, rpt)], acc.at[pl.ds(s * rpt, rpt)])
    pltpu.sync_copy(ones_hbm, ones_v)
    plsc.subcore_barrier()
    base = wid * epw

    def step(j, carry):
      off = base + j * K
      pltpu.sync_copy(dst_hbm.at[pl.ds(off, K)], didx)
      pltpu.sync_copy(ones_v, acc.at[didx], add=True)
      return carry

    lax.fori_loop(0, cpw, step, 0)
    plsc.subcore_barrier()
    pltpu.sync_copy(acc.at[pl.ds(s * rpt, rpt)],
                    out_hbm.at[c, pl.ds(s * rpt, rpt)])

  return body(dst_pad, zeros_d, ones_d)


def _sc_scatter(g, sd2, zeros_d, a_rows, rpt, cpw0, cpw1):
  d = g.shape[1]

  @functools.partial(
      pl.kernel,
      out_type=jax.ShapeDtypeStruct((NC, a_rows, d), jnp.float32),
      mesh=_sc_mesh(),
      scratch_types=[
          pltpu.VMEM((2, K), jnp.int32),
          pltpu.VMEM((K, d), jnp.float32),
          pltpu.VMEM_SHARED((a_rows, d), jnp.float32),
          pltpu.SemaphoreType.DMA,
      ])
  def body(g_hbm, sd_hbm, z_hbm, out_hbm, sdidx, rows, acc, sem):
    c = lax.axis_index("c")
    s = lax.axis_index("s")
    pltpu.sync_copy(z_hbm.at[pl.ds(s * rpt, rpt)], acc.at[pl.ds(s * rpt, rpt)])
    plsc.subcore_barrier()
    cbase = jnp.where(c == 0, s * cpw0, NS * cpw0 + s * cpw1)
    my_cpw = jnp.where(c == 0, cpw0, cpw1)

    def step(j, carry):
      pltpu.sync_copy(sd_hbm.at[cbase + j], sdidx)
      pltpu.async_copy(g_hbm.at[sdidx.at[0]], rows, sem).wait()
      pltpu.sync_copy(rows, acc.at[sdidx.at[1]], add=True)
      return carry

    lax.fori_loop(0, my_cpw, step, 0)
    plsc.subcore_barrier()
    pltpu.sync_copy(acc.at[pl.ds(s * rpt, rpt)],
                    out_hbm.at[c, pl.ds(s * rpt, rpt)])

  return body(g, sd2, zeros_d)


def _tc_first(degp, x, w1, r):
  n, d_in = x.shape
  d_hid = w1.shape[1]
  grid = (n // r,)

  def body(dp_ref, x_ref, w_ref, dinv_ref, g_ref):
    dp = dp_ref[...]
    deg = dp[0, :, 0:1] + dp[1, :, 0:1] + 1.0
    dinv = lax.rsqrt(deg)
    dinv_ref[...] = dinv
    g_ref[...] = dinv * jnp.dot(x_ref[...], w_ref[...],
                                preferred_element_type=jnp.float32)

  return pl.pallas_call(
      body,
      grid=grid,
      in_specs=[
          pl.BlockSpec((NC, r, d_hid), lambda i: (0, i, 0)),
          pl.BlockSpec((r, d_in), lambda i: (i, 0)),
          pl.BlockSpec((d_in, d_hid), lambda i: (0, 0)),
      ],
      out_specs=[
          pl.BlockSpec((r, 1), lambda i: (i, 0)),
          pl.BlockSpec((r, d_hid), lambda i: (i, 0)),
      ],
      out_shape=[
          jax.ShapeDtypeStruct((n, 1), jnp.float32),
          jax.ShapeDtypeStruct((n, d_hid), jnp.float32),
      ])(degp, x, w1)


def _tc_mid(part, g1, dinv, b1, w2, r):
  n, d = g1.shape
  a_rows = part.shape[1]
  grid = (n // r,)

  def body(p_ref, g_ref, dinv_ref, b_ref, w_ref, out_ref):
    p = p_ref[...]
    s = p[0] + p[1] + g_ref[...]
    h = dinv_ref[...] * s + b_ref[...]
    h = jnp.maximum(h, 0.0)
    out_ref[...] = dinv_ref[...] * jnp.dot(h, w_ref[...],
                                           preferred_element_type=jnp.float32)

  return pl.pallas_call(
      body,
      grid=grid,
      in_specs=[
          pl.BlockSpec((NC, r, d), lambda i: (0, i, 0)),
          pl.BlockSpec((r, d), lambda i: (i, 0)),
          pl.BlockSpec((r, 1), lambda i: (i, 0)),
          pl.BlockSpec((1, d), lambda i: (0, 0)),
          pl.BlockSpec((d, d), lambda i: (0, 0)),
      ],
      out_specs=pl.BlockSpec((r, d), lambda i: (i, 0)),
      out_shape=jax.ShapeDtypeStruct((n, d), jnp.float32))(
          part, g1, dinv, b1, w2)


def _tc_last(part, g2, dinv, b2, r):
  n, d = g2.shape
  grid = (n // r,)

  def body(p_ref, g_ref, dinv_ref, b_ref, out_ref):
    p = p_ref[...]
    s = p[0] + p[1] + g_ref[...]
    out_ref[...] = dinv_ref[...] * s + b_ref[...]

  return pl.pallas_call(
      body,
      grid=grid,
      in_specs=[
          pl.BlockSpec((NC, r, d), lambda i: (0, i, 0)),
          pl.BlockSpec((r, d), lambda i: (i, 0)),
          pl.BlockSpec((r, 1), lambda i: (i, 0)),
          pl.BlockSpec((1, d), lambda i: (0, 0)),
      ],
      out_specs=pl.BlockSpec((r, d), lambda i: (i, 0)),
      out_shape=jax.ShapeDtypeStruct((n, d), jnp.float32))(
          part, g2, dinv, b2)


def kernel(x, edge_index, W1, b1, W2, b2):
  n, d_in = x.shape
  d_hid = W1.shape[1]
  e = edge_index.shape[1]

  cpw_deg = -(-e // (NW * K))
  tch = cpw_deg * NW
  e_pad = tch * K
  pad = e_pad - e

  cps = cpw_deg * NC
  cpw0 = (2 * cps) // 3
  cpw1 = cps - cpw0

  rpt = 632
  a_rows = rpt * NS

  src_pad = jnp.concatenate(
      [edge_index[0], jnp.zeros((pad,), jnp.int32)])
  dst_pad = jnp.concatenate(
      [edge_index[1], jnp.full((pad,), n, jnp.int32)])
  sd2 = jnp.stack([src_pad.reshape(-1, K), dst_pad.reshape(-1, K)], axis=1)
  ones_d = jnp.ones((K, d_hid), jnp.float32)
  zeros_d = jnp.zeros((a_rows, d_hid), jnp.float32)

  r = 1000

  degp = _deg_hist(dst_pad, zeros_d, ones_d, a_rows, rpt, cpw_deg,
                   cpw_deg * K, d_hid)
  dinv, g1 = _tc_first(degp, x, W1, r)
  part1 = _sc_scatter(g1, sd2, zeros_d, a_rows, rpt, cpw0, cpw1)
  g2 = _tc_mid(part1, g1, dinv, b1.reshape(1, -1), W2, r)
  part2 = _sc_scatter(g2, sd2, zeros_d, a_rows, rpt, cpw0, cpw1)
  out = _tc_last(part2, g2, dinv, b2.reshape(1, -1), r)
  return out


# speedup vs baseline: 1.7480x; 1.0924x over previous
import functools

import jax
import jax.numpy as jnp
from jax import lax
from jax.experimental import pallas as pl
from jax.experimental.pallas import tpu as pltpu
from jax.experimental.pallas import tpu_sc as plsc

NC = 2
NS = 16
NW = NC * NS
LANES = 16
K = 128


def _sc_mesh():
  return plsc.VectorSubcoreMesh(
      core_axis_name="c", subcore_axis_name="s",
      num_cores=NC, num_subcores=NS)


def _deg_hist(dst_pad, zeros_d, ones_d, a_rows, rpt, cpw, epw, d):
  @functools.partial(
      pl.kernel,
      out_type=jax.ShapeDtypeStruct((NC, a_rows, d), jnp.float32),
      mesh=_sc_mesh(),
      scratch_types=[
          pltpu.VMEM((K,), jnp.int32),
          pltpu.VMEM((K, d), jnp.float32),
          pltpu.VMEM_SHARED((a_rows, d), jnp.float32),
      ])
  def body(dst_hbm, z_hbm, ones_hbm, out_hbm, didx, ones_v, acc):
    c = lax.axis_index("c")
    s = lax.axis_index("s")
    wid = c * NS + s
    pltpu.sync_copy(z_hbm.at[pl.ds(s * rpt, rpt)], acc.at[pl.ds(s * rpt, rpt)])
    pltpu.sync_copy(ones_hbm, ones_v)
    plsc.subcore_barrier()
    base = wid * epw

    def step(j, carry):
      off = base + j * K
      pltpu.sync_copy(dst_hbm.at[pl.ds(off, K)], didx)
      pltpu.sync_copy(ones_v, acc.at[didx], add=True)
      return carry

    lax.fori_loop(0, cpw, step, 0)
    plsc.subcore_barrier()
    pltpu.sync_copy(acc.at[pl.ds(s * rpt, rpt)],
                    out_hbm.at[c, pl.ds(s * rpt, rpt)])

  return body(dst_pad, zeros_d, ones_d)


def _sc_scatter(g, sd2, zeros_d, a_rows, rpt, cpw0, cpw1):
  d = g.shape[1]

  @functools.partial(
      pl.kernel,
      out_type=jax.ShapeDtypeStruct((NC, a_rows, d), jnp.float32),
      mesh=_sc_mesh(),
      scratch_types=[
          pltpu.VMEM((2, K), jnp.int32),
          pltpu.VMEM((2, K), jnp.int32),
          pltpu.VMEM((K, d), jnp.float32),
          pltpu.VMEM((K, d), jnp.float32),
          pltpu.VMEM_SHARED((a_rows, d), jnp.float32),
          pltpu.SemaphoreType.DMA,
          pltpu.SemaphoreType.DMA,
          pltpu.SemaphoreType.DMA,
          pltpu.SemaphoreType.DMA,
      ])
  def body(g_hbm, sd_hbm, z_hbm, out_hbm, sdidx0, sdidx1, rows0, rows1,
           acc, gsem0, gsem1, ssem0, ssem1):
    c = lax.axis_index("c")
    s = lax.axis_index("s")
    pltpu.sync_copy(z_hbm.at[pl.ds(s * rpt, rpt)], acc.at[pl.ds(s * rpt, rpt)])
    plsc.subcore_barrier()
    cbase = jnp.where(c == 0, s * cpw0, NS * cpw0 + s * cpw1)
    my_cpw = jnp.where(c == 0, cpw0, cpw1)

    def step(t, carry):
      cb = cbase + 2 * t
      pltpu.sync_copy(sd_hbm.at[cb], sdidx0)
      g0 = pltpu.async_copy(g_hbm.at[sdidx0.at[0]], rows0, gsem0)
      pltpu.sync_copy(sd_hbm.at[cb + 1], sdidx1)
      g1 = pltpu.async_copy(g_hbm.at[sdidx1.at[0]], rows1, gsem1)
      g0.wait()
      s0 = pltpu.async_copy(rows0, acc.at[sdidx0.at[1]], ssem0, add=True)
      g1.wait()
      s1 = pltpu.async_copy(rows1, acc.at[sdidx1.at[1]], ssem1, add=True)
      s0.wait()
      s1.wait()
      return carry

    lax.fori_loop(0, my_cpw // 2, step, 0)
    plsc.subcore_barrier()
    pltpu.sync_copy(acc.at[pl.ds(s * rpt, rpt)],
                    out_hbm.at[c, pl.ds(s * rpt, rpt)])

  return body(g, sd2, zeros_d)


def _tc_first(degp, x, w1, r):
  n, d_in = x.shape
  d_hid = w1.shape[1]
  grid = (n // r,)

  def body(dp_ref, x_ref, w_ref, dinv_ref, g_ref):
    dp = dp_ref[...]
    deg = dp[0, :, 0:1] + dp[1, :, 0:1] + 1.0
    dinv = lax.rsqrt(deg)
    dinv_ref[...] = dinv
    g_ref[...] = dinv * jnp.dot(x_ref[...], w_ref[...],
                                preferred_element_type=jnp.float32)

  return pl.pallas_call(
      body,
      grid=grid,
      in_specs=[
          pl.BlockSpec((NC, r, d_hid), lambda i: (0, i, 0)),
          pl.BlockSpec((r, d_in), lambda i: (i, 0)),
          pl.BlockSpec((d_in, d_hid), lambda i: (0, 0)),
      ],
      out_specs=[
          pl.BlockSpec((r, 1), lambda i: (i, 0)),
          pl.BlockSpec((r, d_hid), lambda i: (i, 0)),
      ],
      out_shape=[
          jax.ShapeDtypeStruct((n, 1), jnp.float32),
          jax.ShapeDtypeStruct((n, d_hid), jnp.float32),
      ])(degp, x, w1)


def _tc_mid(part, g1, dinv, b1, w2, r):
  n, d = g1.shape
  a_rows = part.shape[1]
  grid = (n // r,)

  def body(p_ref, g_ref, dinv_ref, b_ref, w_ref, out_ref):
    p = p_ref[...]
    s = p[0] + p[1] + g_ref[...]
    h = dinv_ref[...] * s + b_ref[...]
    h = jnp.maximum(h, 0.0)
    out_ref[...] = dinv_ref[...] * jnp.dot(h, w_ref[...],
                                           preferred_element_type=jnp.float32)

  return pl.pallas_call(
      body,
      grid=grid,
      in_specs=[
          pl.BlockSpec((NC, r, d), lambda i: (0, i, 0)),
          pl.BlockSpec((r, d), lambda i: (i, 0)),
          pl.BlockSpec((r, 1), lambda i: (i, 0)),
          pl.BlockSpec((1, d), lambda i: (0, 0)),
          pl.BlockSpec((d, d), lambda i: (0, 0)),
      ],
      out_specs=pl.BlockSpec((r, d), lambda i: (i, 0)),
      out_shape=jax.ShapeDtypeStruct((n, d), jnp.float32))(
          part, g1, dinv, b1, w2)


def _tc_last(part, g2, dinv, b2, r):
  n, d = g2.shape
  grid = (n // r,)

  def body(p_ref, g_ref, dinv_ref, b_ref, out_ref):
    p = p_ref[...]
    s = p[0] + p[1] + g_ref[...]
    out_ref[...] = dinv_ref[...] * s + b_ref[...]

  return pl.pallas_call(
      body,
      grid=grid,
      in_specs=[
          pl.BlockSpec((NC, r, d), lambda i: (0, i, 0)),
          pl.BlockSpec((r, d), lambda i: (i, 0)),
          pl.BlockSpec((r, 1), lambda i: (i, 0)),
          pl.BlockSpec((1, d), lambda i: (0, 0)),
      ],
      out_specs=pl.BlockSpec((r, d), lambda i: (i, 0)),
      out_shape=jax.ShapeDtypeStruct((n, d), jnp.float32))(
          part, g2, dinv, b2)


def kernel(x, edge_index, W1, b1, W2, b2):
  n, d_in = x.shape
  d_hid = W1.shape[1]
  e = edge_index.shape[1]

  cpw_deg = -(-e // (NW * K))
  tch = cpw_deg * NW
  e_pad = tch * K
  pad = e_pad - e

  cps = cpw_deg * NC
  cpw0 = 2 * ((2 * cps // 3 + 1) // 2)
  cpw1 = cps - cpw0
  assert cpw0 % 2 == 0 and cpw1 % 2 == 0

  rpt = 632
  a_rows = rpt * NS

  src_pad = jnp.concatenate(
      [edge_index[0], jnp.zeros((pad,), jnp.int32)])
  dst_pad = jnp.concatenate(
      [edge_index[1], jnp.full((pad,), n, jnp.int32)])
  sd2 = jnp.stack([src_pad.reshape(-1, K), dst_pad.reshape(-1, K)], axis=1)
  ones_d = jnp.ones((K, d_hid), jnp.float32)
  zeros_d = jnp.zeros((a_rows, d_hid), jnp.float32)

  r = 1000

  degp = _deg_hist(dst_pad, zeros_d, ones_d, a_rows, rpt, cpw_deg,
                   cpw_deg * K, d_hid)
  dinv, g1 = _tc_first(degp, x, W1, r)
  part1 = _sc_scatter(g1, sd2, zeros_d, a_rows, rpt, cpw0, cpw1)
  g2 = _tc_mid(part1, g1, dinv, b1.reshape(1, -1), W2, r)
  part2 = _sc_scatter(g2, sd2, zeros_d, a_rows, rpt, cpw0, cpw1)
  out = _tc_last(part2, g2, dinv, b2.reshape(1, -1), r)
  return out


# speedup vs baseline: 1.7537x; 1.0032x over previous
import functools

import jax
import jax.numpy as jnp
from jax import lax
from jax.experimental import pallas as pl
from jax.experimental.pallas import tpu as pltpu
from jax.experimental.pallas import tpu_sc as plsc

NC = 2
NS = 16
NW = NC * NS
LANES = 16
K = 128


def _sc_mesh():
  return plsc.VectorSubcoreMesh(
      core_axis_name="c", subcore_axis_name="s",
      num_cores=NC, num_subcores=NS)


def _deg_hist(dst_pad, zeros_d, ones_d, a_rows, rpt, cpw, epw, d):
  @functools.partial(
      pl.kernel,
      out_type=jax.ShapeDtypeStruct((NC, a_rows, d), jnp.float32),
      mesh=_sc_mesh(),
      scratch_types=[
          pltpu.VMEM((K,), jnp.int32),
          pltpu.VMEM((K, d), jnp.float32),
          pltpu.VMEM_SHARED((a_rows, d), jnp.float32),
      ])
  def body(dst_hbm, z_hbm, ones_hbm, out_hbm, didx, ones_v, acc):
    c = lax.axis_index("c")
    s = lax.axis_index("s")
    wid = c * NS + s
    pltpu.sync_copy(z_hbm.at[pl.ds(s * rpt, rpt)], acc.at[pl.ds(s * rpt, rpt)])
    pltpu.sync_copy(ones_hbm, ones_v)
    plsc.subcore_barrier()
    base = wid * epw

    def step(j, carry):
      off = base + j * K
      pltpu.sync_copy(dst_hbm.at[pl.ds(off, K)], didx)
      pltpu.sync_copy(ones_v, acc.at[didx], add=True)
      return carry

    lax.fori_loop(0, cpw, step, 0)
    plsc.subcore_barrier()
    pltpu.sync_copy(acc.at[pl.ds(s * rpt, rpt)],
                    out_hbm.at[c, pl.ds(s * rpt, rpt)])

  return body(dst_pad, zeros_d, ones_d)


def _sc_scatter(g, sd2, zeros_d, a_rows, rpt, cpw0, cpw1):
  d = g.shape[1]

  @functools.partial(
      pl.kernel,
      out_type=jax.ShapeDtypeStruct((NC, a_rows, d), jnp.float32),
      mesh=_sc_mesh(),
      scratch_types=[
          pltpu.VMEM((2, K), jnp.int32),
          pltpu.VMEM((2, K), jnp.int32),
          pltpu.VMEM((K, d), jnp.float32),
          pltpu.VMEM((K, d), jnp.float32),
          pltpu.VMEM_SHARED((a_rows, d), jnp.float32),
          pltpu.SemaphoreType.DMA,
          pltpu.SemaphoreType.DMA,
          pltpu.SemaphoreType.DMA,
          pltpu.SemaphoreType.DMA,
      ])
  def body(g_hbm, sd_hbm, z_hbm, out_hbm, sdidx0, sdidx1, rows0, rows1,
           acc, gsem0, gsem1, ssem0, ssem1):
    c = lax.axis_index("c")
    s = lax.axis_index("s")
    pltpu.sync_copy(z_hbm.at[pl.ds(s * rpt, rpt)], acc.at[pl.ds(s * rpt, rpt)])
    plsc.subcore_barrier()
    cbase = jnp.where(c == 0, s * cpw0, NS * cpw0 + s * cpw1)
    my_cpw = jnp.where(c == 0, cpw0, cpw1)

    nt = my_cpw // 2

    pltpu.sync_copy(sd_hbm.at[cbase], sdidx0)
    pltpu.async_copy(g_hbm.at[sdidx0.at[0]], rows0, gsem0)

    def step(t, carry):
      cb = cbase + 2 * t
      pltpu.make_async_copy(g_hbm.at[sdidx0.at[0]], rows0, gsem0).wait()
      s0 = pltpu.async_copy(rows0, acc.at[sdidx0.at[1]], ssem0, add=True)

      @pl.when(t > 0)
      def _():
        pltpu.make_async_copy(rows1, acc.at[sdidx1.at[1]], ssem1).wait()

      pltpu.sync_copy(sd_hbm.at[cb + 1], sdidx1)
      g1 = pltpu.async_copy(g_hbm.at[sdidx1.at[0]], rows1, gsem1)
      g1.wait()
      s1 = pltpu.async_copy(rows1, acc.at[sdidx1.at[1]], ssem1, add=True)
      s0.wait()

      @pl.when(t + 1 < nt)
      def _():
        pltpu.sync_copy(sd_hbm.at[cb + 2], sdidx0)
        pltpu.async_copy(g_hbm.at[sdidx0.at[0]], rows0, gsem0)

      return carry

    lax.fori_loop(0, nt, step, 0)
    pltpu.make_async_copy(rows1, acc.at[sdidx1.at[1]], ssem1).wait()
    plsc.subcore_barrier()
    pltpu.sync_copy(acc.at[pl.ds(s * rpt, rpt)],
                    out_hbm.at[c, pl.ds(s * rpt, rpt)])

  return body(g, sd2, zeros_d)


def _tc_first(degp, x, w1, r):
  n, d_in = x.shape
  d_hid = w1.shape[1]
  grid = (n // r,)

  def body(dp_ref, x_ref, w_ref, dinv_ref, g_ref):
    dp = dp_ref[...]
    deg = dp[0, :, 0:1] + dp[1, :, 0:1] + 1.0
    dinv = lax.rsqrt(deg)
    dinv_ref[...] = dinv
    g_ref[...] = dinv * jnp.dot(x_ref[...], w_ref[...],
                                preferred_element_type=jnp.float32)

  return pl.pallas_call(
      body,
      grid=grid,
      in_specs=[
          pl.BlockSpec((NC, r, d_hid), lambda i: (0, i, 0)),
          pl.BlockSpec((r, d_in), lambda i: (i, 0)),
          pl.BlockSpec((d_in, d_hid), lambda i: (0, 0)),
      ],
      out_specs=[
          pl.BlockSpec((r, 1), lambda i: (i, 0)),
          pl.BlockSpec((r, d_hid), lambda i: (i, 0)),
      ],
      out_shape=[
          jax.ShapeDtypeStruct((n, 1), jnp.float32),
          jax.ShapeDtypeStruct((n, d_hid), jnp.float32),
      ])(degp, x, w1)


def _tc_mid(part, g1, dinv, b1, w2, r):
  n, d = g1.shape
  a_rows = part.shape[1]
  grid = (n // r,)

  def body(p_ref, g_ref, dinv_ref, b_ref, w_ref, out_ref):
    p = p_ref[...]
    s = p[0] + p[1] + g_ref[...]
    h = dinv_ref[...] * s + b_ref[...]
    h = jnp.maximum(h, 0.0)
    out_ref[...] = dinv_ref[...] * jnp.dot(h, w_ref[...],
                                           preferred_element_type=jnp.float32)

  return pl.pallas_call(
      body,
      grid=grid,
      in_specs=[
          pl.BlockSpec((NC, r, d), lambda i: (0, i, 0)),
          pl.BlockSpec((r, d), lambda i: (i, 0)),
          pl.BlockSpec((r, 1), lambda i: (i, 0)),
          pl.BlockSpec((1, d), lambda i: (0, 0)),
          pl.BlockSpec((d, d), lambda i: (0, 0)),
      ],
      out_specs=pl.BlockSpec((r, d), lambda i: (i, 0)),
      out_shape=jax.ShapeDtypeStruct((n, d), jnp.float32))(
          part, g1, dinv, b1, w2)


def _tc_last(part, g2, dinv, b2, r):
  n, d = g2.shape
  grid = (n // r,)

  def body(p_ref, g_ref, dinv_ref, b_ref, out_ref):
    p = p_ref[...]
    s = p[0] + p[1] + g_ref[...]
    out_ref[...] = dinv_ref[...] * s + b_ref[...]

  return pl.pallas_call(
      body,
      grid=grid,
      in_specs=[
          pl.BlockSpec((NC, r, d), lambda i: (0, i, 0)),
          pl.BlockSpec((r, d), lambda i: (i, 0)),
          pl.BlockSpec((r, 1), lambda i: (i, 0)),
          pl.BlockSpec((1, d), lambda i: (0, 0)),
      ],
      out_specs=pl.BlockSpec((r, d), lambda i: (i, 0)),
      out_shape=jax.ShapeDtypeStruct((n, d), jnp.float32))(
          part, g2, dinv, b2)


def kernel(x, edge_index, W1, b1, W2, b2):
  n, d_in = x.shape
  d_hid = W1.shape[1]
  e = edge_index.shape[1]

  cpw_deg = -(-e // (NW * K))
  tch = cpw_deg * NW
  e_pad = tch * K
  pad = e_pad - e

  cps = cpw_deg * NC
  cpw0 = 2 * ((2 * cps // 3 + 1) // 2)
  cpw1 = cps - cpw0
  assert cpw0 % 2 == 0 and cpw1 % 2 == 0

  rpt = 632
  a_rows = rpt * NS

  src_pad = jnp.concatenate(
      [edge_index[0], jnp.zeros((pad,), jnp.int32)])
  dst_pad = jnp.concatenate(
      [edge_index[1], jnp.full((pad,), n, jnp.int32)])
  sd2 = jnp.stack([src_pad.reshape(-1, K), dst_pad.reshape(-1, K)], axis=1)
  ones_d = jnp.ones((K, d_hid), jnp.float32)
  zeros_d = jnp.zeros((a_rows, d_hid), jnp.float32)

  r = 1000

  degp = _deg_hist(dst_pad, zeros_d, ones_d, a_rows, rpt, cpw_deg,
                   cpw_deg * K, d_hid)
  dinv, g1 = _tc_first(degp, x, W1, r)
  part1 = _sc_scatter(g1, sd2, zeros_d, a_rows, rpt, cpw0, cpw1)
  g2 = _tc_mid(part1, g1, dinv, b1.reshape(1, -1), W2, r)
  part2 = _sc_scatter(g2, sd2, zeros_d, a_rows, rpt, cpw0, cpw1)
  out = _tc_last(part2, g2, dinv, b2.reshape(1, -1), r)
  return out
